# Initial kernel scaffold; baseline (speedup 1.0000x reference)
import functools

import jax
import jax.numpy as jnp
from jax import lax
from jax.experimental import pallas as pl
from jax.experimental.pallas import tpu as pltpu
from jax.experimental.pallas import tpu_sc as plsc

N = 10000
E = 320000
D = 128
NW = 32
NPW = 313
NPAD = NW * NPW
C = 4000
C16 = C // 16
NCHUNK = E // C
SB = 256
NEG = jnp.float32(-jnp.inf)

_mesh = plsc.VectorSubcoreMesh(core_axis_name="c", subcore_axis_name="s")


@functools.partial(
    pl.kernel,
    out_type=jax.ShapeDtypeStruct((NPAD * D,), jnp.float32),
    mesh=_mesh,
    scratch_types=[
        pltpu.VMEM(((NPW + 1) * D,), jnp.float32),
        pltpu.VMEM((2 * C,), jnp.int32),
        pltpu.VMEM((2 * C,), jnp.int32),
        pltpu.VMEM((C + 16,), jnp.int32),
        pltpu.VMEM((C + 16,), jnp.int32),
        pltpu.VMEM((SB, D), jnp.float32),
        pltpu.SemaphoreType.DMA,
        pltpu.SemaphoreType.DMA,
    ],
)
def _seg_max(x_hbm, src_hbm, dst_hbm, out_hbm,
             agg, srcb, dstb, sel_s, sel_d, rows, esem, rsem):
    wid = lax.axis_index("s") * 2 + lax.axis_index("c")
    lo = wid * NPW
    lane = lax.iota(jnp.int32, 16)

    def init_body(i, _):
        agg[pl.ds(i * 16, 16)] = jnp.full((16,), NEG, jnp.float32)
        return 0
    lax.fori_loop(0, (NPW + 1) * D // 16, init_body, 0)

    pltpu.async_copy(src_hbm.at[pl.ds(0, C)], srcb.at[pl.ds(0, C)], esem)
    pltpu.async_copy(dst_hbm.at[pl.ds(0, C)], dstb.at[pl.ds(0, C)], esem)
    pltpu.async_copy(src_hbm.at[pl.ds(C, C)], srcb.at[pl.ds(C, C)], esem)
    pltpu.async_copy(dst_hbm.at[pl.ds(C, C)], dstb.at[pl.ds(C, C)], esem)

    def chunk_body(cc, _):
        pbase = (cc % 2) * C
        pltpu.make_async_copy(
            src_hbm.at[pl.ds(cc * C, C)], srcb.at[pl.ds(pbase, C)], esem).wait()
        pltpu.make_async_copy(
            dst_hbm.at[pl.ds(cc * C, C)], dstb.at[pl.ds(pbase, C)], esem).wait()

        def fbody(i, cnt):
            s16 = srcb[pl.ds(pbase + i * 16, 16)]
            d16 = dstb[pl.ds(pbase + i * 16, 16)]
            m = (d16 >= lo) & (d16 < lo + NPW)
            plsc.store_compressed(sel_s.at[pl.ds(cnt, 16)], s16, mask=m)
            plsc.store_compressed(sel_d.at[pl.ds(cnt, 16)], d16 - lo, mask=m)
            return cnt + jnp.sum(m.astype(jnp.int32))
        cnt = lax.fori_loop(0, C16, fbody, jnp.int32(0))

        ones = jnp.ones((16,), jnp.bool_)
        plsc.store_compressed(sel_s.at[pl.ds(cnt, 16)],
                              jnp.zeros((16,), jnp.int32), mask=ones)
        plsc.store_compressed(sel_d.at[pl.ds(cnt, 16)],
                              jnp.full((16,), NPW, jnp.int32), mask=ones)

        @pl.when(cc + 2 < NCHUNK)
        def _():
            pltpu.async_copy(src_hbm.at[pl.ds((cc + 2) * C, C)],
                             srcb.at[pl.ds(pbase, C)], esem)
            pltpu.async_copy(dst_hbm.at[pl.ds((cc + 2) * C, C)],
                             dstb.at[pl.ds(pbase, C)], esem)

        ngroups = (cnt + 15) // 16
        nbatch = (ngroups + (SB // 16) - 1) // (SB // 16)

        def bat_body(b, _):
            gbase = b * (SB // 16)
            nb = jnp.minimum(ngroups - gbase, SB // 16)

            def fire(gg, _):
                idx16 = sel_s[pl.ds((gbase + gg) * 16, 16)]
                pltpu.async_copy(x_hbm.at[idx16],
                                 rows.at[pl.ds(gg * 16, 16)], rsem)
                return 0
            lax.fori_loop(0, nb, fire, 0)

            def drain(gg, _):
                pltpu.make_async_copy(x_hbm.at[pl.ds(0, 16)],
                                      rows.at[pl.ds(gg * 16, 16)], rsem).wait()
                return 0
            lax.fori_loop(0, nb, drain, 0)

            def acc(eidx, _):
                e = gbase * 16 + eidx
                ldv = plsc.load_gather(sel_d, [jnp.full((16,), e, jnp.int32)])
                addr0 = ldv * D + lane
                rloc = jnp.full((16,), eidx, jnp.int32)
                for f in range(D // 16):
                    addr = addr0 + f * 16
                    rowv = plsc.load_gather(rows, [rloc, f * 16 + lane])
                    aggv = plsc.load_gather(agg, [addr])
                    plsc.store_scatter(agg, [addr], jnp.maximum(aggv, rowv))
                return 0
            lax.fori_loop(0, nb * 16, acc, 0)
            return 0
        lax.fori_loop(0, nbatch, bat_body, 0)
        return 0

    lax.fori_loop(0, NCHUNK, chunk_body, 0)

    def wb(i, _):
        v = agg[pl.ds(i * 16, 16)]
        agg[pl.ds(i * 16, 16)] = jnp.where(v == NEG, jnp.float32(0.0), v)
        return 0
    lax.fori_loop(0, NPW * D // 16, wb, 0)
    pltpu.sync_copy(agg.at[pl.ds(0, NPW * D)],
                    out_hbm.at[pl.ds(lo * D, NPW * D)])


BR = 2000


def _mm_body(agg_ref, x_ref, wl_ref, wr_ref, b_ref, o_ref, *, elu):
    acc = jnp.dot(agg_ref[...], wl_ref[...], preferred_element_type=jnp.float32)
    acc = acc + jnp.dot(x_ref[...], wr_ref[...],
                        preferred_element_type=jnp.float32)
    acc = acc + b_ref[...]
    if elu:
        acc = jnp.where(acc > 0, acc, jnp.exp(jnp.minimum(acc, 0.0)) - 1.0)
    o_ref[...] = acc


def _mm(agg, x, wl, wr, b, elu):
    body = functools.partial(_mm_body, elu=elu)
    return pl.pallas_call(
        body,
        grid=(N // BR,),
        in_specs=[
            pl.BlockSpec((BR, D), lambda i: (i, 0)),
            pl.BlockSpec((BR, D), lambda i: (i, 0)),
            pl.BlockSpec((D, D), lambda i: (0, 0)),
            pl.BlockSpec((D, D), lambda i: (0, 0)),
            pl.BlockSpec((1, D), lambda i: (0, 0)),
        ],
        out_specs=pl.BlockSpec((BR, D), lambda i: (i, 0)),
        out_shape=jax.ShapeDtypeStruct((N, D), jnp.float32),
    )(agg, x, wl, wr, b)


def kernel(features, edge_index, W_l1, b_l1, W_r1, W_l2, b_l2, W_r2):
    src = edge_index[0]
    dst = edge_index[1]
    agg1 = _seg_max(features, src, dst).reshape(NPAD, D)
    h = _mm(agg1, features, W_l1, W_r1, b_l1.reshape(1, D), elu=True)
    agg2 = _seg_max(h, src, dst).reshape(NPAD, D)
    return _mm(agg2, h, W_l2, W_r2, b_l2.reshape(1, D), elu=False)


# speedup vs baseline: 1.8374x; 1.8374x over previous
import functools

import jax
import jax.numpy as jnp
from jax import lax
from jax.experimental import pallas as pl
from jax.experimental.pallas import tpu as pltpu
from jax.experimental.pallas import tpu_sc as plsc

N = 10000
E = 320000
D = 128
NW = 32
NPW = 313
NPAD = NW * NPW
C = 4000
C16 = C // 16
NCHUNK = E // C
SB = 256
NEG = float("-inf")

_mesh = plsc.VectorSubcoreMesh(core_axis_name="c", subcore_axis_name="s")


@functools.partial(
    pl.kernel,
    out_type=jax.ShapeDtypeStruct((NPAD * D,), jnp.float32),
    mesh=_mesh,
    compiler_params=pltpu.CompilerParams(needs_layout_passes=False),
    scratch_types=[
        pltpu.VMEM(((NPW + 1) * D,), jnp.float32),
        pltpu.VMEM((2 * C,), jnp.int32),
        pltpu.VMEM((2 * C,), jnp.int32),
        pltpu.VMEM((C + 16,), jnp.int32),
        pltpu.VMEM((C + 16,), jnp.int32),
        pltpu.VMEM((SB, D), jnp.float32),
        pltpu.SemaphoreType.DMA,
        pltpu.SemaphoreType.DMA,
    ],
)
def _seg_max(x_hbm, src_hbm, dst_hbm, out_hbm,
             agg, srcb, dstb, sel_s, sel_d, rows, esem, rsem):
    wid = lax.axis_index("s") * 2 + lax.axis_index("c")
    lo = wid * NPW
    lane = lax.iota(jnp.int32, 16)

    def init_body(i, _):
        agg[pl.ds(i * 16, 16)] = jnp.full((16,), NEG, jnp.float32)
        return 0
    lax.fori_loop(0, (NPW + 1) * D // 16, init_body, 0)

    pltpu.async_copy(src_hbm.at[pl.ds(0, C)], srcb.at[pl.ds(0, C)], esem)
    pltpu.async_copy(dst_hbm.at[pl.ds(0, C)], dstb.at[pl.ds(0, C)], esem)
    pltpu.async_copy(src_hbm.at[pl.ds(C, C)], srcb.at[pl.ds(C, C)], esem)
    pltpu.async_copy(dst_hbm.at[pl.ds(C, C)], dstb.at[pl.ds(C, C)], esem)

    def chunk_body(cc, _):
        pbase = (cc % 2) * C
        pltpu.make_async_copy(
            src_hbm.at[pl.ds(cc * C, C)], srcb.at[pl.ds(pbase, C)], esem).wait()
        pltpu.make_async_copy(
            dst_hbm.at[pl.ds(cc * C, C)], dstb.at[pl.ds(pbase, C)], esem).wait()

        def fbody(i, cnt):
            s16 = srcb[pl.ds(pbase + i * 16, 16)]
            d16 = dstb[pl.ds(pbase + i * 16, 16)]
            m = (d16 >= lo) & (d16 < lo + NPW)
            plsc.store_compressed(sel_s.at[pl.ds(cnt, 16)], s16, mask=m)
            plsc.store_compressed(sel_d.at[pl.ds(cnt, 16)], d16 - lo, mask=m)
            return cnt + jnp.sum(m.astype(jnp.int32))
        cnt = lax.fori_loop(0, C16, fbody, jnp.int32(0))

        ones = jnp.ones((16,), jnp.bool_)
        plsc.store_compressed(sel_s.at[pl.ds(cnt, 16)],
                              jnp.zeros((16,), jnp.int32), mask=ones)
        plsc.store_compressed(sel_d.at[pl.ds(cnt, 16)],
                              jnp.full((16,), NPW, jnp.int32), mask=ones)

        @pl.when(cc + 2 < NCHUNK)
        def _():
            pltpu.async_copy(src_hbm.at[pl.ds((cc + 2) * C, C)],
                             srcb.at[pl.ds(pbase, C)], esem)
            pltpu.async_copy(dst_hbm.at[pl.ds((cc + 2) * C, C)],
                             dstb.at[pl.ds(pbase, C)], esem)

        ngroups = (cnt + 15) // 16
        nbatch = (ngroups + (SB // 16) - 1) // (SB // 16)

        def bat_body(b, _):
            gbase = b * (SB // 16)
            nb = jnp.minimum(ngroups - gbase, SB // 16)

            def fire(gg, _):
                idx16 = sel_s[pl.ds((gbase + gg) * 16, 16)]
                pltpu.async_copy(x_hbm.at[idx16],
                                 rows.at[pl.ds(gg * 16, 16)], rsem)
                return 0
            lax.fori_loop(0, nb, fire, 0)

            def drain(gg, _):
                pltpu.make_async_copy(x_hbm.at[pl.ds(0, 16)],
                                      rows.at[pl.ds(gg * 16, 16)], rsem).wait()
                return 0
            lax.fori_loop(0, nb, drain, 0)

            def acc(eidx, _):
                e = gbase * 16 + eidx
                ldv = plsc.load_gather(sel_d, [jnp.full((16,), e, jnp.int32)])
                addr0 = ldv * D + lane
                rloc = jnp.full((16,), eidx, jnp.int32)
                for f in range(D // 16):
                    addr = addr0 + f * 16
                    rowv = plsc.load_gather(rows, [rloc, f * 16 + lane])
                    aggv = plsc.load_gather(agg, [addr])
                    plsc.store_scatter(agg, [addr], jnp.maximum(aggv, rowv))
                return 0
            lax.fori_loop(0, nb * 16, acc, 0)
            return 0
        lax.fori_loop(0, nbatch, bat_body, 0)
        return 0

    lax.fori_loop(0, NCHUNK, chunk_body, 0)

    def wb(i, _):
        v = agg[pl.ds(i * 16, 16)]
        agg[pl.ds(i * 16, 16)] = jnp.where(v == NEG, jnp.float32(0.0), v)
        return 0
    lax.fori_loop(0, NPW * D // 16, wb, 0)
    pltpu.sync_copy(agg.at[pl.ds(0, NPW * D)],
                    out_hbm.at[pl.ds(lo * D, NPW * D)])


BR = 2000


def _mm_body(agg_ref, x_ref, wl_ref, wr_ref, b_ref, o_ref, *, elu):
    acc = jnp.dot(agg_ref[...], wl_ref[...], preferred_element_type=jnp.float32)
    acc = acc + jnp.dot(x_ref[...], wr_ref[...],
                        preferred_element_type=jnp.float32)
    acc = acc + b_ref[...]
    if elu:
        acc = jnp.where(acc > 0, acc, jnp.exp(jnp.minimum(acc, 0.0)) - 1.0)
    o_ref[...] = acc


def _mm(agg, x, wl, wr, b, elu):
    body = functools.partial(_mm_body, elu=elu)
    return pl.pallas_call(
        body,
        grid=(N // BR,),
        in_specs=[
            pl.BlockSpec((BR, D), lambda i: (i, 0)),
            pl.BlockSpec((BR, D), lambda i: (i, 0)),
            pl.BlockSpec((D, D), lambda i: (0, 0)),
            pl.BlockSpec((D, D), lambda i: (0, 0)),
            pl.BlockSpec((1, D), lambda i: (0, 0)),
        ],
        out_specs=pl.BlockSpec((BR, D), lambda i: (i, 0)),
        out_shape=jax.ShapeDtypeStruct((N, D), jnp.float32),
    )(agg, x, wl, wr, b)


def kernel(features, edge_index, W_l1, b_l1, W_r1, W_l2, b_l2, W_r2):
    src = edge_index[0]
    dst = edge_index[1]
    agg1 = _seg_max(features, src, dst).reshape(NPAD, D)
    h = _mm(agg1, features, W_l1, W_r1, b_l1.reshape(1, D), elu=True)
    agg2 = _seg_max(h, src, dst).reshape(NPAD, D)
    return _mm(agg2, h, W_l2, W_r2, b_l2.reshape(1, D), elu=False)


# speedup vs baseline: 2.8040x; 1.5261x over previous
import functools

import jax
import jax.numpy as jnp
from jax import lax
from jax.experimental import pallas as pl
from jax.experimental.pallas import tpu as pltpu
from jax.experimental.pallas import tpu_sc as plsc

N = 10000
E = 320000
D = 128
DP = D // 2
NW = 32
NPW = 313
NPAD = NW * NPW
C = 4000
C16 = C // 16
NCHUNK = E // C
SB = 512
NEGI = -8323200

_mesh = plsc.VectorSubcoreMesh(core_axis_name="c", subcore_axis_name="s")


@functools.partial(
    pl.kernel,
    out_type=jax.ShapeDtypeStruct((NPAD * DP,), jnp.int32),
    mesh=_mesh,
    compiler_params=pltpu.CompilerParams(
        needs_layout_passes=False, use_tc_tiling_on_sc=False),
    scratch_types=[
        pltpu.VMEM(((NPW + 1) * DP,), jnp.int32),
        pltpu.VMEM((2 * C,), jnp.int32),
        pltpu.VMEM((2 * C,), jnp.int32),
        pltpu.VMEM((C + 16,), jnp.int32),
        pltpu.VMEM((C + 16,), jnp.int32),
        pltpu.VMEM((SB, DP), jnp.int32),
        pltpu.SemaphoreType.DMA,
        pltpu.SemaphoreType.DMA,
    ],
)
def _seg_max(x_hbm, src_hbm, dst_hbm, out_hbm,
             agg, srcb, dstb, sel_s, sel_d, rows, esem, rsem):
    wid = lax.axis_index("s") * 2 + lax.axis_index("c")
    lo = wid * NPW
    lane = lax.iota(jnp.int32, 16)

    def init_body(i, _):
        agg[pl.ds(i * 16, 16)] = jnp.full((16,), NEGI, jnp.int32)
        return 0
    lax.fori_loop(0, (NPW + 1) * DP // 16, init_body, 0)

    pltpu.async_copy(src_hbm.at[pl.ds(0, C)], srcb.at[pl.ds(0, C)], esem)
    pltpu.async_copy(dst_hbm.at[pl.ds(0, C)], dstb.at[pl.ds(0, C)], esem)
    pltpu.async_copy(src_hbm.at[pl.ds(C, C)], srcb.at[pl.ds(C, C)], esem)
    pltpu.async_copy(dst_hbm.at[pl.ds(C, C)], dstb.at[pl.ds(C, C)], esem)

    def chunk_body(cc, _):
        pbase = (cc % 2) * C
        pltpu.make_async_copy(
            src_hbm.at[pl.ds(cc * C, C)], srcb.at[pl.ds(pbase, C)], esem).wait()
        pltpu.make_async_copy(
            dst_hbm.at[pl.ds(cc * C, C)], dstb.at[pl.ds(pbase, C)], esem).wait()

        def fbody(i, cntv):
            s16 = srcb[pl.ds(pbase + i * 16, 16)]
            d16 = dstb[pl.ds(pbase + i * 16, 16)]
            m = (d16 >= lo) & (d16 < lo + NPW)
            pos = cntv + plsc.cumsum(m.astype(jnp.int32)) - 1
            plsc.store_scatter(sel_s, [pos], s16, mask=m)
            plsc.store_scatter(sel_d, [pos], d16 - lo, mask=m)
            return cntv + plsc.all_reduce_population_count(m)
        cntv = lax.fori_loop(0, C16, fbody, jnp.zeros((16,), jnp.int32))
        cnt = cntv[0]

        sel_s[pl.ds(cnt, 16)] = jnp.zeros((16,), jnp.int32)
        sel_d[pl.ds(cnt, 16)] = jnp.full((16,), NPW, jnp.int32)

        @pl.when(cc + 2 < NCHUNK)
        def _():
            pltpu.async_copy(src_hbm.at[pl.ds((cc + 2) * C, C)],
                             srcb.at[pl.ds(pbase, C)], esem)
            pltpu.async_copy(dst_hbm.at[pl.ds((cc + 2) * C, C)],
                             dstb.at[pl.ds(pbase, C)], esem)

        ngroups = (cnt + 15) // 16
        nbatch = (ngroups + (SB // 16) - 1) // (SB // 16)

        def bat_body(b, _):
            gbase = b * (SB // 16)
            nb = jnp.minimum(ngroups - gbase, SB // 16)

            def fire(gg, _):
                idx16 = sel_s[pl.ds((gbase + gg) * 16, 16)]
                pltpu.async_copy(x_hbm.at[idx16],
                                 rows.at[pl.ds(gg * 16, 16)], rsem)
                return 0
            lax.fori_loop(0, nb, fire, 0)

            def drain(gg, _):
                pltpu.make_async_copy(x_hbm.at[pl.ds(0, 16)],
                                      rows.at[pl.ds(gg * 16, 16)], rsem).wait()
                return 0
            lax.fori_loop(0, nb, drain, 0)

            def accg(g, _):
                dvec = sel_d[pl.ds((gbase + g) * 16, 16)]
                rbase = g * 16
                for k in range(0, 16, 2):
                    ld0 = dvec[k]
                    ld1 = dvec[k + 1]
                    a0 = jnp.full((16,), ld0 * DP, jnp.int32) + lane
                    a1 = jnp.full((16,), ld1 * DP, jnp.int32) + lane
                    r0 = jnp.full((16,), rbase + k, jnp.int32)
                    r1 = jnp.full((16,), rbase + k + 1, jnp.int32)
                    nc16 = jnp.full((16,), ld0 != ld1)
                    c32 = jnp.full((32,), ld0 == ld1)
                    for f in range(DP // 16):
                        fl = f * 16 + lane
                        rv0 = plsc.load_gather(rows, [r0, fl])
                        av0 = plsc.load_gather(agg, [a0 + f * 16])
                        rv1 = plsc.load_gather(rows, [r1, fl])
                        av1 = plsc.load_gather(agg, [a1 + f * 16])
                        rb0 = plsc.bitcast(rv0, jnp.bfloat16)
                        rb1 = plsc.bitcast(rv1, jnp.bfloat16)
                        m0 = jnp.maximum(plsc.bitcast(av0, jnp.bfloat16), rb0)
                        m1 = jnp.maximum(plsc.bitcast(av1, jnp.bfloat16), rb1)
                        m1 = jnp.where(c32, jnp.maximum(m1, rb0), m1)
                        plsc.store_scatter(agg, [a0 + f * 16],
                                           plsc.bitcast(m0, jnp.int32),
                                           mask=nc16)
                        plsc.store_scatter(agg, [a1 + f * 16],
                                           plsc.bitcast(m1, jnp.int32))
                return 0
            lax.fori_loop(0, nb, accg, 0)
            return 0
        lax.fori_loop(0, nbatch, bat_body, 0)
        return 0

    lax.fori_loop(0, NCHUNK, chunk_body, 0)

    def wb(i, _):
        v = plsc.bitcast(agg[pl.ds(i * 16, 16)], jnp.bfloat16)
        v = jnp.where(v == jnp.bfloat16(float("-inf")), jnp.bfloat16(0), v)
        agg[pl.ds(i * 16, 16)] = plsc.bitcast(v, jnp.int32)
        return 0
    lax.fori_loop(0, NPW * DP // 16, wb, 0)
    pltpu.sync_copy(agg.at[pl.ds(0, NPW * DP)],
                    out_hbm.at[pl.ds(lo * DP, NPW * DP)])


BR = 2000


def _mm_body(agg_ref, x_ref, wl_ref, wr_ref, b_ref, o_ref, *, elu):
    a = agg_ref[...].astype(jnp.float32)
    acc = jnp.dot(a, wl_ref[...], preferred_element_type=jnp.float32)
    acc = acc + jnp.dot(x_ref[...], wr_ref[...],
                        preferred_element_type=jnp.float32)
    acc = acc + b_ref[...]
    if elu:
        acc = jnp.where(acc > 0, acc, jnp.exp(jnp.minimum(acc, 0.0)) - 1.0)
    o_ref[...] = acc


def _mm(agg, x, wl, wr, b, elu):
    body = functools.partial(_mm_body, elu=elu)
    return pl.pallas_call(
        body,
        grid=(N // BR,),
        in_specs=[
            pl.BlockSpec((BR, D), lambda i: (i, 0)),
            pl.BlockSpec((BR, D), lambda i: (i, 0)),
            pl.BlockSpec((D, D), lambda i: (0, 0)),
            pl.BlockSpec((D, D), lambda i: (0, 0)),
            pl.BlockSpec((1, D), lambda i: (0, 0)),
        ],
        out_specs=pl.BlockSpec((BR, D), lambda i: (i, 0)),
        out_shape=jax.ShapeDtypeStruct((N, D), jnp.float32),
    )(agg, x, wl, wr, b)


def _pack(x):
    return jax.lax.bitcast_convert_type(
        x.astype(jnp.bfloat16).reshape(x.shape[0], DP, 2), jnp.int32)


def _unpack(p):
    return jax.lax.bitcast_convert_type(
        p.reshape(-1, DP), jnp.bfloat16).reshape(-1, D)


def kernel(features, edge_index, W_l1, b_l1, W_r1, W_l2, b_l2, W_r2):
    src = edge_index[0]
    dst = edge_index[1]
    agg1 = _unpack(_seg_max(_pack(features), src, dst))
    h = _mm(agg1, features, W_l1, W_r1, b_l1.reshape(1, D), elu=True)
    agg2 = _unpack(_seg_max(_pack(h), src, dst))
    return _mm(agg2, h, W_l2, W_r2, b_l2.reshape(1, D), elu=False)


# speedup vs baseline: 3.2680x; 1.1655x over previous
import functools

import jax
import jax.numpy as jnp
from jax import lax
from jax.experimental import pallas as pl
from jax.experimental.pallas import tpu as pltpu
from jax.experimental.pallas import tpu_sc as plsc

N = 10000
E = 320000
D = 128
DP = D // 2
NW = 32
NPW = 313
NPAD = NW * NPW
C = 4000
C16 = C // 16
NCHUNK = E // C
SEL = C + 16
SB = 512
GPB = SB // 16
NEGI = -8323200

_mesh = plsc.VectorSubcoreMesh(core_axis_name="c", subcore_axis_name="s")


@functools.partial(
    pl.kernel,
    out_type=jax.ShapeDtypeStruct((NPAD * DP,), jnp.int32),
    mesh=_mesh,
    compiler_params=pltpu.CompilerParams(
        needs_layout_passes=False, use_tc_tiling_on_sc=False),
    scratch_types=[
        pltpu.VMEM(((NPW + 1) * DP,), jnp.int32),
        pltpu.VMEM((2 * C,), jnp.int32),
        pltpu.VMEM((2 * C,), jnp.int32),
        pltpu.VMEM((2 * SEL,), jnp.int32),
        pltpu.VMEM((2 * SEL,), jnp.int32),
        pltpu.VMEM((2 * SB, DP), jnp.int32),
        pltpu.SemaphoreType.DMA,
        pltpu.SemaphoreType.DMA,
        pltpu.SemaphoreType.DMA,
    ],
)
def _seg_max(x_hbm, src_hbm, dst_hbm, out_hbm,
             agg, srcb, dstb, sel_s, sel_d, rows, esem, rsem0, rsem1):
    wid = lax.axis_index("s") * 2 + lax.axis_index("c")
    lo = wid * NPW
    lane = lax.iota(jnp.int32, 16)

    def init_body(i, _):
        agg[pl.ds(i * 16, 16)] = jnp.full((16,), NEGI, jnp.int32)
        return 0
    lax.fori_loop(0, (NPW + 1) * DP // 16, init_body, 0)

    def filter_chunk(cc, par):
        pbase = par * C
        sbase = par * SEL
        pltpu.make_async_copy(
            src_hbm.at[pl.ds(cc * C, C)], srcb.at[pl.ds(pbase, C)], esem).wait()
        pltpu.make_async_copy(
            dst_hbm.at[pl.ds(cc * C, C)], dstb.at[pl.ds(pbase, C)], esem).wait()

        def fbody(i, cntv):
            s16 = srcb[pl.ds(pbase + i * 16, 16)]
            d16 = dstb[pl.ds(pbase + i * 16, 16)]
            m = (d16 >= lo) & (d16 < lo + NPW)
            pos = cntv + plsc.cumsum(m.astype(jnp.int32)) - 1 + sbase
            plsc.store_scatter(sel_s, [pos], s16, mask=m)
            plsc.store_scatter(sel_d, [pos], d16 - lo, mask=m)
            return cntv + plsc.all_reduce_population_count(m)
        cntv = lax.fori_loop(0, C16, fbody, jnp.zeros((16,), jnp.int32))
        cnt = cntv[0]
        sel_s[pl.ds(sbase + cnt, 16)] = jnp.zeros((16,), jnp.int32)
        sel_d[pl.ds(sbase + cnt, 16)] = jnp.full((16,), NPW, jnp.int32)

        @pl.when(cc + 2 < NCHUNK)
        def _():
            pltpu.async_copy(src_hbm.at[pl.ds((cc + 2) * C, C)],
                             srcb.at[pl.ds(pbase, C)], esem)
            pltpu.async_copy(dst_hbm.at[pl.ds((cc + 2) * C, C)],
                             dstb.at[pl.ds(pbase, C)], esem)
        return cnt

    def fire_groups(par, rsem, gbase, ng):
        sbase = par * SEL
        rbase = par * SB

        def fire(gg, _):
            idx16 = sel_s[pl.ds(sbase + (gbase + gg) * 16, 16)]
            pltpu.async_copy(x_hbm.at[idx16],
                             rows.at[pl.ds(rbase + gg * 16, 16)], rsem)
            return 0
        lax.fori_loop(0, ng, fire, 0)

    def drain_groups(par, rsem, ng):
        rbase = par * SB

        def drain(gg, _):
            pltpu.make_async_copy(x_hbm.at[pl.ds(0, 16)],
                                  rows.at[pl.ds(rbase + gg * 16, 16)],
                                  rsem).wait()
            return 0
        lax.fori_loop(0, ng, drain, 0)

    def acc_groups(par, gbase, ng):
        sbase = par * SEL
        rbase = par * SB

        def accg(g, _):
            dvec = sel_d[pl.ds(sbase + (gbase + g) * 16, 16)]
            rrow = rbase + g * 16
            for k in range(0, 16, 2):
                ld0 = dvec[k]
                ld1 = dvec[k + 1]
                a0 = jnp.full((16,), ld0 * DP, jnp.int32) + lane
                a1 = jnp.full((16,), ld1 * DP, jnp.int32) + lane
                r0 = jnp.full((16,), rrow + k, jnp.int32)
                r1 = jnp.full((16,), rrow + k + 1, jnp.int32)
                nc16 = jnp.full((16,), ld0 != ld1)
                c32 = jnp.full((32,), ld0 == ld1)
                for f in range(DP // 16):
                    fl = f * 16 + lane
                    rv0 = plsc.load_gather(rows, [r0, fl])
                    av0 = plsc.load_gather(agg, [a0 + f * 16])
                    rv1 = plsc.load_gather(rows, [r1, fl])
                    av1 = plsc.load_gather(agg, [a1 + f * 16])
                    rb0 = plsc.bitcast(rv0, jnp.bfloat16)
                    rb1 = plsc.bitcast(rv1, jnp.bfloat16)
                    m0 = jnp.maximum(plsc.bitcast(av0, jnp.bfloat16), rb0)
                    m1 = jnp.maximum(plsc.bitcast(av1, jnp.bfloat16), rb1)
                    m1 = jnp.where(c32, jnp.maximum(m1, rb0), m1)
                    plsc.store_scatter(agg, [a0 + f * 16],
                                       plsc.bitcast(m0, jnp.int32), mask=nc16)
                    plsc.store_scatter(agg, [a1 + f * 16],
                                       plsc.bitcast(m1, jnp.int32))
            return 0
        lax.fori_loop(0, ng, accg, 0)

    def acc_chunk(cnt, par, rsem):
        ngroups = (cnt + 15) // 16
        nb0 = jnp.minimum(ngroups, GPB)
        drain_groups(par, rsem, nb0)
        acc_groups(par, 0, nb0)
        nbatch = (ngroups + GPB - 1) // GPB

        def lb(b, _):
            gbase = b * GPB
            nb = jnp.minimum(ngroups - gbase, GPB)
            fire_groups(par, rsem, gbase, nb)
            drain_groups(par, rsem, nb)
            acc_groups(par, gbase, nb)
            return 0
        lax.fori_loop(1, nbatch, lb, 0)

    pltpu.async_copy(src_hbm.at[pl.ds(0, C)], srcb.at[pl.ds(0, C)], esem)
    pltpu.async_copy(dst_hbm.at[pl.ds(0, C)], dstb.at[pl.ds(0, C)], esem)
    pltpu.async_copy(src_hbm.at[pl.ds(C, C)], srcb.at[pl.ds(C, C)], esem)
    pltpu.async_copy(dst_hbm.at[pl.ds(C, C)], dstb.at[pl.ds(C, C)], esem)
    cnt0 = filter_chunk(0, 0)
    fire_groups(0, rsem0, 0, jnp.minimum((cnt0 + 15) // 16, GPB))

    def pair_body(i, pcnt):
        c1 = 2 * i + 1
        cnt1 = filter_chunk(c1, 1)
        fire_groups(1, rsem1, 0, jnp.minimum((cnt1 + 15) // 16, GPB))
        acc_chunk(pcnt, 0, rsem0)
        cnt2 = filter_chunk(c1 + 1, 0)
        fire_groups(0, rsem0, 0, jnp.minimum((cnt2 + 15) // 16, GPB))
        acc_chunk(cnt1, 1, rsem1)
        return cnt2

    pcnt = lax.fori_loop(0, (NCHUNK - 2) // 2, pair_body, cnt0)

    cntl = filter_chunk(NCHUNK - 1, 1)
    fire_groups(1, rsem1, 0, jnp.minimum((cntl + 15) // 16, GPB))
    acc_chunk(pcnt, 0, rsem0)
    acc_chunk(cntl, 1, rsem1)

    def wb(i, _):
        v = plsc.bitcast(agg[pl.ds(i * 16, 16)], jnp.bfloat16)
        v = jnp.where(v == jnp.bfloat16(float("-inf")), jnp.bfloat16(0), v)
        agg[pl.ds(i * 16, 16)] = plsc.bitcast(v, jnp.int32)
        return 0
    lax.fori_loop(0, NPW * DP // 16, wb, 0)
    pltpu.sync_copy(agg.at[pl.ds(0, NPW * DP)],
                    out_hbm.at[pl.ds(lo * DP, NPW * DP)])


BR = 2000


def _mm_body(agg_ref, x_ref, wl_ref, wr_ref, b_ref, o_ref, *, elu):
    a = agg_ref[...].astype(jnp.float32)
    acc = jnp.dot(a, wl_ref[...], preferred_element_type=jnp.float32)
    acc = acc + jnp.dot(x_ref[...], wr_ref[...],
                        preferred_element_type=jnp.float32)
    acc = acc + b_ref[...]
    if elu:
        acc = jnp.where(acc > 0, acc, jnp.exp(jnp.minimum(acc, 0.0)) - 1.0)
    o_ref[...] = acc


def _mm(agg, x, wl, wr, b, elu):
    body = functools.partial(_mm_body, elu=elu)
    return pl.pallas_call(
        body,
        grid=(N // BR,),
        in_specs=[
            pl.BlockSpec((BR, D), lambda i: (i, 0)),
            pl.BlockSpec((BR, D), lambda i: (i, 0)),
            pl.BlockSpec((D, D), lambda i: (0, 0)),
            pl.BlockSpec((D, D), lambda i: (0, 0)),
            pl.BlockSpec((1, D), lambda i: (0, 0)),
        ],
        out_specs=pl.BlockSpec((BR, D), lambda i: (i, 0)),
        out_shape=jax.ShapeDtypeStruct((N, D), jnp.float32),
    )(agg, x, wl, wr, b)


def _pack(x):
    return jax.lax.bitcast_convert_type(
        x.astype(jnp.bfloat16).reshape(x.shape[0], DP, 2), jnp.int32)


def _unpack(p):
    return jax.lax.bitcast_convert_type(
        p.reshape(-1, DP), jnp.bfloat16).reshape(-1, D)


def kernel(features, edge_index, W_l1, b_l1, W_r1, W_l2, b_l2, W_r2):
    src = edge_index[0]
    dst = edge_index[1]
    agg1 = _unpack(_seg_max(_pack(features), src, dst))
    h = _mm(agg1, features, W_l1, W_r1, b_l1.reshape(1, D), elu=True)
    agg2 = _unpack(_seg_max(_pack(h), src, dst))
    return _mm(agg2, h, W_l2, W_r2, b_l2.reshape(1, D), elu=False)
